# baseline (speedup 1.0000x reference)
import jax
import jax.numpy as jnp
from jax.experimental import pallas as pl
from jax.experimental.pallas import tpu as pltpu


def _shift_body(x4_ref, x_ref, mask_ref, out_ref, stage_ref, sem1, sem2):
    i = pl.program_id(0)
    c2 = x_ref.shape[1]
    ch = c2 // 2
    h, w = mask_ref.shape
    n = h * w

    cp1 = pltpu.make_async_copy(x4_ref.at[i], out_ref.at[i, pl.ds(0, c2)],
                                sem1)
    cp1.start()

    x = x_ref[0]
    fmr = x[:ch]
    lat = x[ch:]
    mrow = jnp.concatenate([mask_ref[r:r + 1, :] for r in range(h)],
                           axis=1)
    frow = mrow >= 1
    fcol = mrow.T >= 1

    lat_t = lat.T
    norm = jnp.sqrt(jnp.sum(lat_t * lat_t, axis=1, keepdims=True)) + 1e-8
    latn = lat_t / norm
    sim = jax.lax.dot_general(
        latn, latn, (((1,), (1,)), ((), ())),
        preferred_element_type=jnp.float32,
        precision=jax.lax.Precision.DEFAULT)
    sim = jnp.where(frow, jnp.float32(-1e9), sim)
    rowmax = jnp.max(sim, axis=1, keepdims=True)
    kiota = jax.lax.broadcasted_iota(jnp.int32, (n, n), 1)
    idx = jnp.min(jnp.where(sim == rowmax, kiota, n), axis=1,
                  keepdims=True)
    niota = jax.lax.broadcasted_iota(jnp.int32, (n, 1), 0)
    sel = jnp.where(fcol, idx, niota)
    onehot = (sel == kiota).astype(jnp.bfloat16)
    f0 = fmr.astype(jnp.bfloat16)
    r1 = fmr - f0.astype(jnp.float32)
    f1 = r1.astype(jnp.bfloat16)
    f2 = (r1 - f1.astype(jnp.float32)).astype(jnp.bfloat16)

    def _pass(f):
        return jax.lax.dot_general(
            f, onehot, (((1,), (1,)), ((), ())),
            preferred_element_type=jnp.float32)

    shifted = (_pass(f0) + _pass(f1)) + _pass(f2)
    for r in range(h):
        stage_ref[:, r, :] = shifted[:, r * w:(r + 1) * w]
    cp2 = pltpu.make_async_copy(stage_ref, out_ref.at[i, pl.ds(c2, ch)],
                                sem2)
    cp2.start()
    cp1.wait()
    cp2.wait()


def kernel(input, mask):
    b, c, h, w = input.shape
    ch = c // 2
    n = h * w
    x = input.reshape(b, c, n)

    out = pl.pallas_call(
        _shift_body,
        grid=(b,),
        in_specs=[
            pl.BlockSpec(memory_space=pltpu.MemorySpace.HBM),
            pl.BlockSpec((1, c, n), lambda i: (i, 0, 0)),
            pl.BlockSpec((h, w), lambda i: (0, 0)),
        ],
        out_specs=pl.BlockSpec(memory_space=pltpu.MemorySpace.HBM),
        out_shape=jax.ShapeDtypeStruct((b, c + ch, h, w), jnp.float32),
        scratch_shapes=[
            pltpu.VMEM((ch, h, w), jnp.float32),
            pltpu.SemaphoreType.DMA,
            pltpu.SemaphoreType.DMA,
        ],
    )(input, x, mask)

    return out


# speedup vs baseline: 22.1862x; 22.1862x over previous
import jax
import jax.numpy as jnp
from jax.experimental import pallas as pl


def _shift_body(x_ref, mask_ref, out_ref):
    x = x_ref[0]
    c2 = x.shape[0]
    ch = c2 // 2
    h, w = mask_ref.shape
    n = h * w
    fmr = x[:ch]
    lat = x[ch:]
    mrow = jnp.concatenate([mask_ref[r:r + 1, :] for r in range(h)],
                           axis=1)
    frow = mrow >= 1
    fcol = mrow.T >= 1

    lat_t = lat.T
    norm = jnp.sqrt(jnp.sum(lat_t * lat_t, axis=1, keepdims=True)) + 1e-8
    latn = lat_t / norm
    sim = jax.lax.dot_general(
        latn, latn, (((1,), (1,)), ((), ())),
        preferred_element_type=jnp.float32,
        precision=jax.lax.Precision.DEFAULT)
    sim = jnp.where(frow, jnp.float32(-1e9), sim)
    rowmax = jnp.max(sim, axis=1, keepdims=True)
    kiota = jax.lax.broadcasted_iota(jnp.int32, (n, n), 1)
    idx = jnp.min(jnp.where(sim == rowmax, kiota, n), axis=1,
                  keepdims=True)
    niota = jax.lax.broadcasted_iota(jnp.int32, (n, 1), 0)
    sel = jnp.where(fcol, idx, niota)
    onehot = (sel == kiota).astype(jnp.bfloat16)
    f0 = fmr.astype(jnp.bfloat16)
    r1 = fmr - f0.astype(jnp.float32)
    f1 = r1.astype(jnp.bfloat16)
    f2 = (r1 - f1.astype(jnp.float32)).astype(jnp.bfloat16)

    def _pass(f):
        return jax.lax.dot_general(
            f, onehot, (((1,), (1,)), ((), ())),
            preferred_element_type=jnp.float32)

    shifted = (_pass(f0) + _pass(f1)) + _pass(f2)
    out_ref[0, :c2] = x
    out_ref[0, c2:] = shifted


def kernel(input, mask):
    b, c, h, w = input.shape
    ch = c // 2
    n = h * w
    x = input.reshape(b, c, n)

    out = pl.pallas_call(
        _shift_body,
        grid=(b,),
        in_specs=[
            pl.BlockSpec((1, c, n), lambda i: (i, 0, 0)),
            pl.BlockSpec((h, w), lambda i: (0, 0)),
        ],
        out_specs=pl.BlockSpec((1, c + ch, n), lambda i: (i, 0, 0)),
        out_shape=jax.ShapeDtypeStruct((b, c + ch, n), jnp.float32),
    )(x, mask)

    return out.reshape(b, c + ch, h, w)
